# Initial kernel scaffold; baseline (speedup 1.0000x reference)
import jax
import jax.numpy as jnp
from jax.experimental import pallas as pl


def kernel(x, rms_weight, gate_w, fc1_w, fc1_b, fc2_w, fc2_b):
    raise NotImplementedError("write your pallas kernel here")


# speedup vs baseline: 2.6114x; 2.6114x over previous
import functools

import jax
import jax.numpy as jnp
from jax.experimental import pallas as pl
from jax.experimental.pallas import tpu as pltpu

D_MODEL = 1024
HIDDEN = 4096
NUM_EXPERTS = 16
TOP_K = 2
EPS = 1e-6

BT = 1024
HB = 2048


def _moe_dense_kernel(x_ref, rmsw_ref, gate_ref, fc1_ref, fc1b_ref,
                      fc2_ref, fc2b_ref, out_ref, tok_bf, tw, acc):
    e = pl.program_id(1)
    h = pl.program_id(2)

    @pl.when(jnp.logical_and(e == 0, h == 0))
    def _prologue():
        x = x_ref[...]
        t = x * jax.lax.rsqrt(
            jnp.mean(jnp.square(x), axis=-1, keepdims=True) + EPS)
        t = t * rmsw_ref[...][None, :]
        logits = jnp.dot(t, gate_ref[...].T,
                         preferred_element_type=jnp.float32)
        tok_bf[...] = t.astype(jnp.bfloat16)
        m0 = jnp.max(logits, axis=-1, keepdims=True)
        i0 = jnp.argmax(logits, axis=-1)
        col = jax.lax.broadcasted_iota(jnp.int32, logits.shape, 1)
        masked = jnp.where(col == i0[:, None], -jnp.inf, logits)
        m1 = jnp.max(masked, axis=-1, keepdims=True)
        i1 = jnp.argmax(masked, axis=-1)
        w0 = 1.0 / (1.0 + jnp.exp(m1 - m0))
        w1 = 1.0 - w0
        tw[...] = (w0 * (col == i0[:, None]) + w1 * (col == i1[:, None]))
        acc[...] = jnp.zeros_like(acc)

    t_bf = tok_bf[...]
    hid = jnp.dot(t_bf, fc1_ref[0].T, preferred_element_type=jnp.float32)
    hid = hid + fc1b_ref[0]
    hid = 0.5 * hid * (1.0 + jax.lax.erf(hid * 0.7071067811865476))
    part = jnp.dot(hid.astype(jnp.bfloat16), fc2_ref[0].T,
                   preferred_element_type=jnp.float32)
    twf = tw[...]
    ecol = jax.lax.broadcasted_iota(jnp.int32, twf.shape, 1)
    tw_e = jnp.sum(jnp.where(ecol == e, twf, 0.0), axis=1, keepdims=True)
    contrib = tw_e * part

    @pl.when(h == 0)
    def _add_bias():
        acc[...] += tw_e * fc2b_ref[0]

    acc[...] += contrib

    @pl.when(jnp.logical_and(e == NUM_EXPERTS - 1, h == HIDDEN // HB - 1))
    def _epilogue():
        out_ref[...] = x_ref[...] + acc[...]


def kernel(x, rms_weight, gate_w, fc1_w, fc1_b, fc2_w, fc2_b):
    b, s, d = x.shape
    n_tok = b * s
    xf = x.reshape(n_tok, d)
    fc1_bf = fc1_w.astype(jnp.bfloat16)
    fc2_bf = fc2_w.astype(jnp.bfloat16)
    fc1_b3 = fc1_b.reshape(NUM_EXPERTS, 1, HIDDEN)
    fc2_b3 = fc2_b.reshape(NUM_EXPERTS, 1, D_MODEL)

    grid = (n_tok // BT, NUM_EXPERTS, HIDDEN // HB)
    out = pl.pallas_call(
        _moe_dense_kernel,
        grid=grid,
        in_specs=[
            pl.BlockSpec((BT, d), lambda t, e, h: (t, 0)),
            pl.BlockSpec((d,), lambda t, e, h: (0,)),
            pl.BlockSpec((NUM_EXPERTS, d), lambda t, e, h: (0, 0)),
            pl.BlockSpec((1, HB, d), lambda t, e, h: (e, h, 0)),
            pl.BlockSpec((1, 1, HB), lambda t, e, h: (e, 0, h)),
            pl.BlockSpec((1, d, HB), lambda t, e, h: (e, 0, h)),
            pl.BlockSpec((1, 1, d), lambda t, e, h: (e, 0, 0)),
        ],
        out_specs=pl.BlockSpec((BT, d), lambda t, e, h: (t, 0)),
        out_shape=jax.ShapeDtypeStruct((n_tok, d), jnp.float32),
        scratch_shapes=[
            pltpu.VMEM((BT, d), jnp.bfloat16),
            pltpu.VMEM((BT, NUM_EXPERTS), jnp.float32),
            pltpu.VMEM((BT, d), jnp.float32),
        ],
    )(xf, rms_weight, gate_w, fc1_bf, fc1_b3, fc2_bf, fc2_b3)
    return out.reshape(b, s, d)


# speedup vs baseline: 3.3880x; 1.2974x over previous
import functools

import jax
import jax.numpy as jnp
from jax import lax
from jax.experimental import pallas as pl
from jax.experimental.pallas import tpu as pltpu
from jax.experimental.pallas import tpu_sc as plsc

D_MODEL = 1024
HIDDEN = 4096
NUM_EXPERTS = 16
EPS = 1e-6

N_TOK = 8192
N_PAIR = 2 * N_TOK
BT = 1024
TM = 128
P_SLOTS = N_PAIR + NUM_EXPERTS * TM
NT = P_SLOTS // TM

NW = 32
G_CHUNK = 96
C_CHUNK = 64


def _route_kernel(x_ref, rmsw_ref, gate_ref, tok_ref, eidx_ref, w_ref,
                  rank_ref, counts_ref, base_ref, te_ref, run_cnt):
    t = pl.program_id(0)

    @pl.when(t == 0)
    def _init():
        run_cnt[...] = jnp.zeros_like(run_cnt)

    x = x_ref[...]
    nrm = x * lax.rsqrt(jnp.mean(jnp.square(x), axis=-1, keepdims=True) + EPS)
    nrm = nrm * rmsw_ref[...][None, :]
    tok_ref[...] = nrm.astype(jnp.bfloat16)

    logits = jnp.dot(nrm, gate_ref[...].T, preferred_element_type=jnp.float32)
    m0 = jnp.max(logits, axis=-1, keepdims=True)
    i0 = jnp.argmax(logits, axis=-1).astype(jnp.int32)
    col = lax.broadcasted_iota(jnp.int32, logits.shape, 1)
    masked = jnp.where(col == i0[:, None], -jnp.inf, logits)
    m1 = jnp.max(masked, axis=-1, keepdims=True)
    i1 = jnp.argmax(masked, axis=-1).astype(jnp.int32)
    w0 = 1.0 / (1.0 + jnp.exp(m1 - m0))
    w1 = 1.0 - w0
    eidx_ref[...] = jnp.concatenate([i0[:, None], i1[:, None]], axis=1)
    w_ref[...] = jnp.concatenate([w0, w1], axis=1)

    oh0 = (col == i0[:, None]).astype(jnp.float32)
    oh1 = (col == i1[:, None]).astype(jnp.float32)
    c = oh0 + oh1
    row = lax.broadcasted_iota(jnp.int32, (BT, BT), 0)
    cidx = lax.broadcasted_iota(jnp.int32, (BT, BT), 1)
    lt = (cidx < row).astype(jnp.float32)
    excl = jnp.dot(lt, c, preferred_element_type=jnp.float32)
    run = run_cnt[...].astype(jnp.float32)
    tot = excl + run
    rank0 = jnp.sum(tot * oh0, axis=1, keepdims=True)
    rank1 = jnp.sum(tot * oh1, axis=1, keepdims=True)
    rank_ref[...] = jnp.concatenate([rank0, rank1], axis=1).astype(jnp.int32)
    run_cnt[...] += jnp.sum(c, axis=0, keepdims=True).astype(jnp.int32)

    cnt = run_cnt[...]
    counts_ref[...] = cnt
    padded = ((cnt + (TM - 1)) // TM) * TM
    pr = lax.broadcasted_iota(jnp.int32, (NUM_EXPERTS, NUM_EXPERTS), 0)
    pc = lax.broadcasted_iota(jnp.int32, (NUM_EXPERTS, NUM_EXPERTS), 1)
    contrib = jnp.where(pr < pc, jnp.broadcast_to(
        padded.reshape(NUM_EXPERTS, 1), (NUM_EXPERTS, NUM_EXPERTS)), 0)
    base = jnp.sum(contrib, axis=0).reshape(1, NUM_EXPERTS)
    base_ref[...] = base
    ends = (base + padded).reshape(NUM_EXPERTS, 1)
    jt = lax.broadcasted_iota(jnp.int32, (NUM_EXPERTS, NT), 1) * TM
    te = jnp.sum((ends <= jt).astype(jnp.int32), axis=0).reshape(1, NT)
    te_ref[...] = jnp.minimum(te, NUM_EXPERTS - 1)


def _route(xf, rms_weight, gate_w):
    return pl.pallas_call(
        _route_kernel,
        grid=(N_TOK // BT,),
        in_specs=[
            pl.BlockSpec((BT, D_MODEL), lambda t: (t, 0)),
            pl.BlockSpec((D_MODEL,), lambda t: (0,)),
            pl.BlockSpec((NUM_EXPERTS, D_MODEL), lambda t: (0, 0)),
        ],
        out_specs=[
            pl.BlockSpec((BT, D_MODEL), lambda t: (t, 0)),
            pl.BlockSpec((BT, 2), lambda t: (t, 0)),
            pl.BlockSpec((BT, 2), lambda t: (t, 0)),
            pl.BlockSpec((BT, 2), lambda t: (t, 0)),
            pl.BlockSpec((1, NUM_EXPERTS), lambda t: (0, 0)),
            pl.BlockSpec((1, NUM_EXPERTS), lambda t: (0, 0)),
            pl.BlockSpec((1, NT), lambda t: (0, 0)),
        ],
        out_shape=[
            jax.ShapeDtypeStruct((N_TOK, D_MODEL), jnp.bfloat16),
            jax.ShapeDtypeStruct((N_TOK, 2), jnp.int32),
            jax.ShapeDtypeStruct((N_TOK, 2), jnp.float32),
            jax.ShapeDtypeStruct((N_TOK, 2), jnp.int32),
            jax.ShapeDtypeStruct((1, NUM_EXPERTS), jnp.int32),
            jax.ShapeDtypeStruct((1, NUM_EXPERTS), jnp.int32),
            jax.ShapeDtypeStruct((1, NT), jnp.int32),
        ],
        scratch_shapes=[pltpu.VMEM((1, NUM_EXPERTS), jnp.int32)],
    )(xf, rms_weight, gate_w)


def _dispatch_body(eidx_hbm, rank_hbm, w_hbm, base_hbm,
                   pos_hbm, gtok_hbm, gw_hbm,
                   ve, vr, vw, vbase, vpos, vgtok, vgw):
    cid = lax.axis_index("c")
    sid = lax.axis_index("s")

    @pl.when(jnp.logical_and(cid == 0, sid == 0))
    def _work():
        pltpu.sync_copy(eidx_hbm, ve)
        pltpu.sync_copy(rank_hbm, vr)
        pltpu.sync_copy(w_hbm, vw)
        pltpu.sync_copy(base_hbm, vbase)

        zi = jnp.zeros((16,), jnp.int32)
        zf = jnp.zeros((16,), jnp.float32)

        def zero_body(i, _):
            vgtok[pl.ds(i * 16, 16)] = zi
            vgw[pl.ds(i * 16, 16)] = zf
            return 0

        lax.fori_loop(0, P_SLOTS // 16, zero_body, 0)

        lane = lax.iota(jnp.int32, 16)

        def body(g, _):
            ev = ve[pl.ds(g * 16, 16)]
            rr = vr[pl.ds(g * 16, 16)]
            ww = vw[pl.ds(g * 16, 16)]
            bb = plsc.load_gather(vbase, [ev])
            pos = rr + bb
            vpos[pl.ds(g * 16, 16)] = pos
            tokid = jnp.right_shift(g * 16 + lane, 1)
            plsc.store_scatter(vgtok, [pos], tokid)
            plsc.store_scatter(vgw, [pos], ww)
            return 0

        lax.fori_loop(0, N_PAIR // 16, body, 0)

        pltpu.sync_copy(vpos, pos_hbm)
        pltpu.sync_copy(vgtok, gtok_hbm)
        pltpu.sync_copy(vgw, gw_hbm)


def _dispatch(eidx_f, rank_f, w_f, base_f):
    mesh = plsc.VectorSubcoreMesh(core_axis_name="c", subcore_axis_name="s")
    fn = pl.kernel(
        _dispatch_body,
        out_type=[
            jax.ShapeDtypeStruct((N_PAIR,), jnp.int32),
            jax.ShapeDtypeStruct((P_SLOTS,), jnp.int32),
            jax.ShapeDtypeStruct((P_SLOTS,), jnp.float32),
        ],
        mesh=mesh,
        scratch_types=[
            pltpu.VMEM((N_PAIR,), jnp.int32),
            pltpu.VMEM((N_PAIR,), jnp.int32),
            pltpu.VMEM((N_PAIR,), jnp.float32),
            pltpu.VMEM((NUM_EXPERTS,), jnp.int32),
            pltpu.VMEM((N_PAIR,), jnp.int32),
            pltpu.VMEM((P_SLOTS,), jnp.int32),
            pltpu.VMEM((P_SLOTS,), jnp.float32),
        ],
        compiler_params=pltpu.CompilerParams(needs_layout_passes=False),
    )
    return fn(eidx_f, rank_f, w_f, base_f)


def _make_gather(n_rows, n_chunk, dtype, sl=8):
    rows_per_w = n_rows // NW
    n_loops = rows_per_w // n_chunk
    assert rows_per_w % n_chunk == 0

    def body(idx_hbm, table_hbm, out_hbm, idx_v, rows_v, sem):
        cid = lax.axis_index("c")
        sid = lax.axis_index("s")
        wid = sid * 2 + cid
        base = wid * rows_per_w
        pltpu.sync_copy(idx_hbm.at[pl.ds(base, rows_per_w)], idx_v)
        for ci in range(n_loops):
            idx_slice = idx_v.at[pl.ds(ci * n_chunk, n_chunk)]
            pltpu.async_copy(table_hbm.at[idx_slice], rows_v, sem).wait()
            pltpu.sync_copy(rows_v,
                            out_hbm.at[pl.ds(base + ci * n_chunk, n_chunk)])

    mesh = plsc.VectorSubcoreMesh(core_axis_name="c", subcore_axis_name="s")
    return pl.kernel(
        body,
        out_type=jax.ShapeDtypeStruct((n_rows, sl, 128), dtype),
        mesh=mesh,
        scratch_types=[
            pltpu.VMEM((rows_per_w,), jnp.int32),
            pltpu.VMEM((n_chunk, sl, 128), dtype),
            pltpu.SemaphoreType.DMA,
        ],
        compiler_params=pltpu.CompilerParams(needs_layout_passes=False),
    )


def _ffn_kernel(te_ref, gtok_ref, gw_ref, fc1_ref, fc1b_ref, fc2_ref,
                fc2b_ref, out_ref):
    toks = gtok_ref[...]
    hid = jnp.dot(toks, fc1_ref[0].T, preferred_element_type=jnp.float32)
    hid = hid + fc1b_ref[0]
    hid = 0.5 * hid * (1.0 + lax.erf(hid * 0.7071067811865476))
    o = jnp.dot(hid.astype(jnp.bfloat16), fc2_ref[0].T,
                preferred_element_type=jnp.float32)
    o = o + fc2b_ref[0]
    out_ref[...] = o * gw_ref[...]


def _ffn(te, gtoks, gw, fc1_bf, fc1_b3, fc2_bf, fc2_b3):
    grid_spec = pltpu.PrefetchScalarGridSpec(
        num_scalar_prefetch=1,
        grid=(NT,),
        in_specs=[
            pl.BlockSpec((TM, D_MODEL), lambda t, te: (t, 0)),
            pl.BlockSpec((TM, 1), lambda t, te: (t, 0)),
            pl.BlockSpec((1, HIDDEN, D_MODEL), lambda t, te: (te[t], 0, 0)),
            pl.BlockSpec((1, 1, HIDDEN), lambda t, te: (te[t], 0, 0)),
            pl.BlockSpec((1, D_MODEL, HIDDEN), lambda t, te: (te[t], 0, 0)),
            pl.BlockSpec((1, 1, D_MODEL), lambda t, te: (te[t], 0, 0)),
        ],
        out_specs=pl.BlockSpec((TM, D_MODEL), lambda t, te: (t, 0)),
    )
    return pl.pallas_call(
        _ffn_kernel,
        grid_spec=grid_spec,
        out_shape=jax.ShapeDtypeStruct((P_SLOTS, D_MODEL), jnp.float32),
    )(te, gtoks, gw, fc1_bf, fc1_b3, fc2_bf, fc2_b3)


def _combine_kernel(x_ref, op_ref, out_ref):
    out_ref[...] = x_ref[...] + op_ref[:, 0, :] + op_ref[:, 1, :]


def _combine(xf, op_tok3):
    return pl.pallas_call(
        _combine_kernel,
        grid=(N_TOK // BT,),
        in_specs=[
            pl.BlockSpec((BT, D_MODEL), lambda t: (t, 0)),
            pl.BlockSpec((BT, 2, D_MODEL), lambda t: (t, 0, 0)),
        ],
        out_specs=pl.BlockSpec((BT, D_MODEL), lambda t: (t, 0)),
        out_shape=jax.ShapeDtypeStruct((N_TOK, D_MODEL), jnp.float32),
    )(xf, op_tok3)


def kernel(x, rms_weight, gate_w, fc1_w, fc1_b, fc2_w, fc2_b):
    b, s, d = x.shape
    xf = x.reshape(N_TOK, d)
    fc1_bf = fc1_w.astype(jnp.bfloat16)
    fc2_bf = fc2_w.astype(jnp.bfloat16)
    fc1_b3 = fc1_b.reshape(NUM_EXPERTS, 1, HIDDEN)
    fc2_b3 = fc2_b.reshape(NUM_EXPERTS, 1, D_MODEL)

    tok_bf, eidx, wpair, rank, _counts, base, te = _route(
        xf, rms_weight, gate_w)

    pos, gtok, gw = _dispatch(eidx.reshape(N_PAIR), rank.reshape(N_PAIR),
                              wpair.reshape(N_PAIR), base.reshape(NUM_EXPERTS))

    tok_i32 = lax.bitcast_convert_type(
        tok_bf.reshape(N_TOK, D_MODEL // 2, 2), jnp.int32)
    gtoks3 = _make_gather(P_SLOTS, G_CHUNK, jnp.int32, sl=4)(
        gtok, tok_i32.reshape(N_TOK, 4, 128))
    gtoks_bf = lax.bitcast_convert_type(
        gtoks3.reshape(P_SLOTS, D_MODEL // 2), jnp.bfloat16)

    out_pairs = _ffn(te.reshape(NT), gtoks_bf.reshape(P_SLOTS, D_MODEL),
                     gw.reshape(P_SLOTS, 1), fc1_bf, fc1_b3, fc2_bf, fc2_b3)

    op_tok = _make_gather(N_PAIR, C_CHUNK, jnp.float32)(
        pos, out_pairs.reshape(P_SLOTS, 8, 128))

    out = _combine(xf, op_tok.reshape(N_TOK, 2, D_MODEL))
    return out.reshape(b, s, d)


# speedup vs baseline: 5.2086x; 1.5373x over previous
import functools

import jax
import jax.numpy as jnp
from jax import lax
from jax.experimental import pallas as pl
from jax.experimental.pallas import tpu as pltpu
from jax.experimental.pallas import tpu_sc as plsc

D_MODEL = 1024
HIDDEN = 4096
NUM_EXPERTS = 16
EPS = 1e-6

N_TOK = 8192
N_PAIR = 2 * N_TOK
BT = 1024
TM = 256
P_SLOTS = N_PAIR + NUM_EXPERTS * TM
NT = P_SLOTS // TM

NW = 32
G_CHUNK = 80
C_CHUNK = 64


def _route_kernel(x_ref, rmsw_ref, gate_ref, tok_ref, eidx_ref, w_ref,
                  rank_ref, counts_ref, base_ref, te_ref, run_cnt):
    t = pl.program_id(0)

    @pl.when(t == 0)
    def _init():
        run_cnt[...] = jnp.zeros_like(run_cnt)

    x = x_ref[...]
    nrm = x * lax.rsqrt(jnp.mean(jnp.square(x), axis=-1, keepdims=True) + EPS)
    nrm = nrm * rmsw_ref[...][None, :]
    tok_ref[...] = nrm

    logits = jnp.dot(nrm, gate_ref[...].T, preferred_element_type=jnp.float32)
    m0 = jnp.max(logits, axis=-1, keepdims=True)
    i0 = jnp.argmax(logits, axis=-1).astype(jnp.int32)
    col = lax.broadcasted_iota(jnp.int32, logits.shape, 1)
    masked = jnp.where(col == i0[:, None], -jnp.inf, logits)
    m1 = jnp.max(masked, axis=-1, keepdims=True)
    i1 = jnp.argmax(masked, axis=-1).astype(jnp.int32)
    w0 = 1.0 / (1.0 + jnp.exp(m1 - m0))
    w1 = 1.0 - w0
    eidx_ref[...] = jnp.concatenate([i0[:, None], i1[:, None]], axis=1)
    w_ref[...] = jnp.concatenate([w0, w1], axis=1)

    oh0 = (col == i0[:, None]).astype(jnp.float32)
    oh1 = (col == i1[:, None]).astype(jnp.float32)
    c = oh0 + oh1
    row = lax.broadcasted_iota(jnp.int32, (BT, BT), 0)
    cidx = lax.broadcasted_iota(jnp.int32, (BT, BT), 1)
    lt = (cidx < row).astype(jnp.float32)
    excl = jnp.dot(lt, c, preferred_element_type=jnp.float32)
    run = run_cnt[...].astype(jnp.float32)
    tot = excl + run
    rank0 = jnp.sum(tot * oh0, axis=1, keepdims=True)
    rank1 = jnp.sum(tot * oh1, axis=1, keepdims=True)
    rank_ref[...] = jnp.concatenate([rank0, rank1], axis=1).astype(jnp.int32)
    run_cnt[...] += jnp.sum(c, axis=0, keepdims=True).astype(jnp.int32)

    cnt = run_cnt[...]
    counts_ref[...] = cnt
    padded = ((cnt + (TM - 1)) // TM) * TM
    pr = lax.broadcasted_iota(jnp.int32, (NUM_EXPERTS, NUM_EXPERTS), 0)
    pc = lax.broadcasted_iota(jnp.int32, (NUM_EXPERTS, NUM_EXPERTS), 1)
    contrib = jnp.where(pr < pc, jnp.broadcast_to(
        padded.reshape(NUM_EXPERTS, 1), (NUM_EXPERTS, NUM_EXPERTS)), 0)
    base = jnp.sum(contrib, axis=0).reshape(1, NUM_EXPERTS)
    base_ref[...] = base
    ends = (base + padded).reshape(NUM_EXPERTS, 1)
    jt = lax.broadcasted_iota(jnp.int32, (NUM_EXPERTS, NT), 1) * TM
    te = jnp.sum((ends <= jt).astype(jnp.int32), axis=0).reshape(1, NT)
    te_ref[...] = jnp.minimum(te, NUM_EXPERTS - 1)


def _route(xf, rms_weight, gate_w):
    return pl.pallas_call(
        _route_kernel,
        grid=(N_TOK // BT,),
        in_specs=[
            pl.BlockSpec((BT, D_MODEL), lambda t: (t, 0)),
            pl.BlockSpec((D_MODEL,), lambda t: (0,)),
            pl.BlockSpec((NUM_EXPERTS, D_MODEL), lambda t: (0, 0)),
        ],
        out_specs=[
            pl.BlockSpec((BT, D_MODEL), lambda t: (t, 0)),
            pl.BlockSpec((BT, 2), lambda t: (t, 0)),
            pl.BlockSpec((BT, 2), lambda t: (t, 0)),
            pl.BlockSpec((BT, 2), lambda t: (t, 0)),
            pl.BlockSpec((1, NUM_EXPERTS), lambda t: (0, 0)),
            pl.BlockSpec((1, NUM_EXPERTS), lambda t: (0, 0)),
            pl.BlockSpec((1, NT), lambda t: (0, 0)),
        ],
        out_shape=[
            jax.ShapeDtypeStruct((N_TOK, D_MODEL), jnp.float32),
            jax.ShapeDtypeStruct((N_TOK, 2), jnp.int32),
            jax.ShapeDtypeStruct((N_TOK, 2), jnp.float32),
            jax.ShapeDtypeStruct((N_TOK, 2), jnp.int32),
            jax.ShapeDtypeStruct((1, NUM_EXPERTS), jnp.int32),
            jax.ShapeDtypeStruct((1, NUM_EXPERTS), jnp.int32),
            jax.ShapeDtypeStruct((1, NT), jnp.int32),
        ],
        scratch_shapes=[pltpu.VMEM((1, NUM_EXPERTS), jnp.int32)],
    )(xf, rms_weight, gate_w)


def _dispatch_body(eidx_hbm, rank_hbm, w_hbm, base_hbm,
                   pos_hbm, gtok_hbm, gw_hbm,
                   ve, vr, vw, vbase, vpos, vgtok, vgw):
    cid = lax.axis_index("c")
    sid = lax.axis_index("s")

    @pl.when(jnp.logical_and(cid == 0, sid == 0))
    def _work():
        pltpu.sync_copy(eidx_hbm, ve)
        pltpu.sync_copy(rank_hbm, vr)
        pltpu.sync_copy(w_hbm, vw)
        pltpu.sync_copy(base_hbm, vbase)

        zi = jnp.zeros((16,), jnp.int32)
        zf = jnp.zeros((16,), jnp.float32)

        def zero_body(i, _):
            vgtok[pl.ds(i * 16, 16)] = zi
            vgw[pl.ds(i * 16, 16)] = zf
            return 0

        lax.fori_loop(0, P_SLOTS // 16, zero_body, 0)

        lane = lax.iota(jnp.int32, 16)

        def body(g, _):
            ev = ve[pl.ds(g * 16, 16)]
            rr = vr[pl.ds(g * 16, 16)]
            ww = vw[pl.ds(g * 16, 16)]
            bb = plsc.load_gather(vbase, [ev])
            pos = rr + bb
            vpos[pl.ds(g * 16, 16)] = pos
            tokid = jnp.right_shift(g * 16 + lane, 1)
            plsc.store_scatter(vgtok, [pos], tokid)
            plsc.store_scatter(vgw, [pos], ww)
            return 0

        lax.fori_loop(0, N_PAIR // 16, body, 0)

        pltpu.sync_copy(vpos, pos_hbm)
        pltpu.sync_copy(vgtok, gtok_hbm)
        pltpu.sync_copy(vgw, gw_hbm)


def _dispatch(eidx_f, rank_f, w_f, base_f):
    mesh = plsc.VectorSubcoreMesh(core_axis_name="c", subcore_axis_name="s")
    fn = pl.kernel(
        _dispatch_body,
        out_type=[
            jax.ShapeDtypeStruct((N_PAIR,), jnp.int32),
            jax.ShapeDtypeStruct((P_SLOTS,), jnp.int32),
            jax.ShapeDtypeStruct((P_SLOTS,), jnp.float32),
        ],
        mesh=mesh,
        scratch_types=[
            pltpu.VMEM((N_PAIR,), jnp.int32),
            pltpu.VMEM((N_PAIR,), jnp.int32),
            pltpu.VMEM((N_PAIR,), jnp.float32),
            pltpu.VMEM((NUM_EXPERTS,), jnp.int32),
            pltpu.VMEM((N_PAIR,), jnp.int32),
            pltpu.VMEM((P_SLOTS,), jnp.int32),
            pltpu.VMEM((P_SLOTS,), jnp.float32),
        ],
        compiler_params=pltpu.CompilerParams(needs_layout_passes=False),
    )
    return fn(eidx_f, rank_f, w_f, base_f)


def _make_gather(n_rows, n_chunk, dtype, sl=8):
    rows_per_w = n_rows // NW
    n_loops = rows_per_w // n_chunk
    assert rows_per_w % n_chunk == 0

    def body(idx_hbm, table_hbm, out_hbm, idx_v, rows_v, sem):
        cid = lax.axis_index("c")
        sid = lax.axis_index("s")
        wid = sid * 2 + cid
        base = wid * rows_per_w
        pltpu.sync_copy(idx_hbm.at[pl.ds(base, rows_per_w)], idx_v)
        for ci in range(n_loops):
            idx_slice = idx_v.at[pl.ds(ci * n_chunk, n_chunk)]
            pltpu.async_copy(table_hbm.at[idx_slice], rows_v, sem).wait()
            pltpu.sync_copy(rows_v,
                            out_hbm.at[pl.ds(base + ci * n_chunk, n_chunk)])

    mesh = plsc.VectorSubcoreMesh(core_axis_name="c", subcore_axis_name="s")
    return pl.kernel(
        body,
        out_type=jax.ShapeDtypeStruct((n_rows, sl, 128), dtype),
        mesh=mesh,
        scratch_types=[
            pltpu.VMEM((rows_per_w,), jnp.int32),
            pltpu.VMEM((n_chunk, sl, 128), dtype),
            pltpu.SemaphoreType.DMA,
        ],
        compiler_params=pltpu.CompilerParams(needs_layout_passes=False),
    )


def _cast_kernel(fc1_ref, fc2_ref, o1_ref, o2_ref):
    o1_ref[...] = fc1_ref[...].astype(jnp.bfloat16)
    o2_ref[...] = fc2_ref[...].astype(jnp.bfloat16)


def _cast_weights(fc1_w, fc2_w):
    nh = HIDDEN // D_MODEL
    return pl.pallas_call(
        _cast_kernel,
        grid=(NUM_EXPERTS, nh),
        in_specs=[
            pl.BlockSpec((1, D_MODEL, D_MODEL), lambda e, h: (e, h, 0)),
            pl.BlockSpec((1, D_MODEL, D_MODEL), lambda e, h: (e, 0, h)),
        ],
        out_specs=[
            pl.BlockSpec((1, D_MODEL, D_MODEL), lambda e, h: (e, h, 0)),
            pl.BlockSpec((1, D_MODEL, D_MODEL), lambda e, h: (e, 0, h)),
        ],
        out_shape=[
            jax.ShapeDtypeStruct((NUM_EXPERTS, HIDDEN, D_MODEL), jnp.bfloat16),
            jax.ShapeDtypeStruct((NUM_EXPERTS, D_MODEL, HIDDEN), jnp.bfloat16),
        ],
    )(fc1_w, fc2_w)


def _ffn_kernel(te_ref, gtok_ref, gw_ref, fc1_ref, fc1b_ref, fc2_ref,
                fc2b_ref, out_ref):
    toks = gtok_ref[...].astype(jnp.bfloat16)
    hid = jnp.dot(toks, fc1_ref[0].T, preferred_element_type=jnp.float32)
    hid = hid + fc1b_ref[0]
    hid = 0.5 * hid * (1.0 + lax.erf(hid * 0.7071067811865476))
    o = jnp.dot(hid.astype(jnp.bfloat16), fc2_ref[0].T,
                preferred_element_type=jnp.float32)
    o = o + fc2b_ref[0]
    out_ref[...] = o * gw_ref[...]


def _ffn(te, gtoks, gw, fc1_bf, fc1_b3, fc2_bf, fc2_b3):
    grid_spec = pltpu.PrefetchScalarGridSpec(
        num_scalar_prefetch=1,
        grid=(NT,),
        in_specs=[
            pl.BlockSpec((TM, D_MODEL), lambda t, te: (t, 0)),
            pl.BlockSpec((TM, 1), lambda t, te: (t, 0)),
            pl.BlockSpec((1, HIDDEN, D_MODEL), lambda t, te: (te[t], 0, 0)),
            pl.BlockSpec((1, 1, HIDDEN), lambda t, te: (te[t], 0, 0)),
            pl.BlockSpec((1, D_MODEL, HIDDEN), lambda t, te: (te[t], 0, 0)),
            pl.BlockSpec((1, 1, D_MODEL), lambda t, te: (te[t], 0, 0)),
        ],
        out_specs=pl.BlockSpec((TM, D_MODEL), lambda t, te: (t, 0)),
    )
    return pl.pallas_call(
        _ffn_kernel,
        grid_spec=grid_spec,
        out_shape=jax.ShapeDtypeStruct((P_SLOTS, D_MODEL), jnp.float32),
    )(te, gtoks, gw, fc1_bf, fc1_b3, fc2_bf, fc2_b3)


def _combine_kernel(x_ref, op_ref, out_ref):
    out_ref[...] = x_ref[...] + op_ref[:, 0, :] + op_ref[:, 1, :]


def _combine(xf, op_tok3):
    return pl.pallas_call(
        _combine_kernel,
        grid=(N_TOK // BT,),
        in_specs=[
            pl.BlockSpec((BT, D_MODEL), lambda t: (t, 0)),
            pl.BlockSpec((BT, 2, D_MODEL), lambda t: (t, 0, 0)),
        ],
        out_specs=pl.BlockSpec((BT, D_MODEL), lambda t: (t, 0)),
        out_shape=jax.ShapeDtypeStruct((N_TOK, D_MODEL), jnp.float32),
    )(xf, op_tok3)


def kernel(x, rms_weight, gate_w, fc1_w, fc1_b, fc2_w, fc2_b):
    b, s, d = x.shape
    xf = x.reshape(N_TOK, d)
    fc1_bf, fc2_bf = _cast_weights(fc1_w, fc2_w)
    fc1_b3 = fc1_b.reshape(NUM_EXPERTS, 1, HIDDEN)
    fc2_b3 = fc2_b.reshape(NUM_EXPERTS, 1, D_MODEL)

    tok_f, eidx, wpair, rank, _counts, base, te = _route(
        xf, rms_weight, gate_w)

    pos, gtok, gw = _dispatch(eidx.reshape(N_PAIR), rank.reshape(N_PAIR),
                              wpair.reshape(N_PAIR), base.reshape(NUM_EXPERTS))

    gtoks3 = _make_gather(P_SLOTS, G_CHUNK, jnp.float32, sl=8)(
        gtok, tok_f.reshape(N_TOK, 8, 128))

    out_pairs = _ffn(te.reshape(NT), gtoks3.reshape(P_SLOTS, D_MODEL),
                     gw.reshape(P_SLOTS, 1), fc1_bf, fc1_b3, fc2_bf, fc2_b3)

    op_tok = _make_gather(N_PAIR, C_CHUNK, jnp.float32)(
        pos, out_pairs.reshape(P_SLOTS, 8, 128))

    out = _combine(xf, op_tok.reshape(N_TOK, 2, D_MODEL))
    return out.reshape(b, s, d)
